# Initial kernel scaffold; baseline (speedup 1.0000x reference)
import jax
import jax.numpy as jnp
from jax.experimental import pallas as pl


def kernel(history_item_ids, history_item_masks, embedding_table, code_book, W_enc, b_enc):
    raise NotImplementedError("write your pallas kernel here")


# speedup vs baseline: 1.3492x; 1.3492x over previous
import functools

import jax
import jax.numpy as jnp
from jax import lax
from jax.experimental import pallas as pl
from jax.experimental.pallas import tpu as pltpu
from jax.experimental.pallas import tpu_sc as plsc

D = 128
K = 8192
L = 20
B = 1024
N = B * L

NC = 2
NS = 16
NW = NC * NS
ROWS_PER_W = N // NW
IDX_CHUNK = 128
N_CHUNKS = ROWS_PER_W // IDX_CHUNK
OUT_PER_W = B // NW

_sc_mesh = plsc.VectorSubcoreMesh(core_axis_name="c", subcore_axis_name="s")


@functools.partial(
    pl.kernel,
    mesh=_sc_mesh,
    out_type=jax.ShapeDtypeStruct((N, D), jnp.float32),
    scratch_types=[
        pltpu.VMEM((ROWS_PER_W,), jnp.int32),
        pltpu.VMEM((ROWS_PER_W, D), jnp.float32),
        pltpu.SemaphoreType.DMA,
    ],
)
def _sc_gather_rows(table_hbm, idx_hbm, out_hbm, idx_v, rows_v, sem):
    wid = lax.axis_index("s") * NC + lax.axis_index("c")
    pltpu.sync_copy(idx_hbm.at[pl.ds(wid * ROWS_PER_W, ROWS_PER_W)], idx_v)
    cps = [
        pltpu.async_copy(
            table_hbm.at[idx_v.at[pl.ds(c * IDX_CHUNK, IDX_CHUNK)]],
            rows_v.at[pl.ds(c * IDX_CHUNK, IDX_CHUNK)],
            sem,
        )
        for c in range(N_CHUNKS)
    ]
    for cp in cps:
        cp.wait()
    pltpu.sync_copy(rows_v, out_hbm.at[pl.ds(wid * ROWS_PER_W, ROWS_PER_W)])


@functools.partial(
    pl.kernel,
    mesh=_sc_mesh,
    out_type=jax.ShapeDtypeStruct((B, D), jnp.float32),
    scratch_types=[
        pltpu.VMEM((ROWS_PER_W,), jnp.int32),
        pltpu.VMEM((ROWS_PER_W, D), jnp.float32),
        pltpu.VMEM((OUT_PER_W, D), jnp.float32),
        pltpu.SemaphoreType.DMA,
    ],
)
def _sc_quantize_sum(cb_hbm, idx_hbm, out_hbm, idx_v, rows_v, out_v, sem):
    wid = lax.axis_index("s") * NC + lax.axis_index("c")
    pltpu.sync_copy(idx_hbm.at[pl.ds(wid * ROWS_PER_W, ROWS_PER_W)], idx_v)
    cps = [
        pltpu.async_copy(
            cb_hbm.at[idx_v.at[pl.ds(c * IDX_CHUNK, IDX_CHUNK)]],
            rows_v.at[pl.ds(c * IDX_CHUNK, IDX_CHUNK)],
            sem,
        )
        for c in range(N_CHUNKS)
    ]
    for cp in cps:
        cp.wait()

    def body(b, carry):
        t0 = b * L
        accs = [jnp.zeros((16,), jnp.float32) for _ in range(D // 16)]
        for l in range(L):
            for c in range(D // 16):
                accs[c] = accs[c] + rows_v[t0 + l, pl.ds(c * 16, 16)]
        for c in range(D // 16):
            out_v[b, pl.ds(c * 16, 16)] = accs[c]
        return carry

    lax.fori_loop(0, OUT_PER_W, body, 0)
    pltpu.sync_copy(out_v, out_hbm.at[pl.ds(wid * OUT_PER_W, OUT_PER_W)])


def _tc_finalize(sums_ref, mask_ref, out_ref):
    cnt = jnp.sum((mask_ref[...] >= 1).astype(jnp.float32), axis=1,
                  keepdims=True)
    out_ref[...] = sums_ref[...] / cnt


_tc_finalize_call = pl.pallas_call(
    _tc_finalize,
    in_specs=[
        pl.BlockSpec((B, D), lambda: (0, 0)),
        pl.BlockSpec((B, L), lambda: (0, 0)),
    ],
    out_specs=pl.BlockSpec((B, D), lambda: (0, 0)),
    out_shape=jax.ShapeDtypeStruct((B, D), jnp.float32),
)


TOK_TILE = 256
N_TILES = N // TOK_TILE


def _tc_encode(hist_ref, w_ref, b_ref, out_ref):
    out_ref[...] = jnp.dot(hist_ref[...], w_ref[...],
                           preferred_element_type=jnp.float32) + b_ref[...]


_tc_encode_call = pl.pallas_call(
    _tc_encode,
    grid=(N_TILES,),
    in_specs=[
        pl.BlockSpec((TOK_TILE, D), lambda i: (i, 0)),
        pl.BlockSpec((D, D), lambda i: (0, 0)),
        pl.BlockSpec((1, D), lambda i: (0, 0)),
    ],
    out_specs=pl.BlockSpec((TOK_TILE, D), lambda i: (i, 0)),
    out_shape=jax.ShapeDtypeStruct((N, D), jnp.float32),
    compiler_params=pltpu.CompilerParams(
        dimension_semantics=("arbitrary",),
    ),
)


def _tc_argmin(x_ref, xsq_ref, csq_ref, cb_ref, out_ref):
    x = x_ref[...]
    mm = lax.dot_general(x, cb_ref[...], (((1,), (1,)), ((), ())),
                         preferred_element_type=jnp.float32)
    dist = (xsq_ref[...] + csq_ref[...]) - 2.0 * mm
    minv = jnp.min(dist, axis=1, keepdims=True)
    kiota = lax.broadcasted_iota(jnp.int32, (TOK_TILE, K), 1)
    idx = jnp.min(jnp.where(dist == minv, kiota, K), axis=1, keepdims=True)
    out_ref[...] = idx


_tc_argmin_call = pl.pallas_call(
    _tc_argmin,
    grid=(N_TILES,),
    in_specs=[
        pl.BlockSpec((TOK_TILE, D), lambda i: (i, 0)),
        pl.BlockSpec((TOK_TILE, 1), lambda i: (i, 0)),
        pl.BlockSpec((1, K), lambda i: (0, 0)),
        pl.BlockSpec((K, D), lambda i: (0, 0)),
    ],
    out_specs=pl.BlockSpec((TOK_TILE, 1), lambda i: (i, 0)),
    out_shape=jax.ShapeDtypeStruct((N, 1), jnp.int32),
    compiler_params=pltpu.CompilerParams(
        dimension_semantics=("arbitrary",),
    ),
)


def kernel(history_item_ids, history_item_masks, embedding_table,
           code_book, W_enc, b_enc):
    ids_flat = history_item_ids.reshape(N)
    hist = _sc_gather_rows(embedding_table, ids_flat)
    x = _tc_encode_call(hist.astype(jnp.bfloat16), W_enc, b_enc.reshape(1, D))
    xsq = jnp.sum(x ** 2, axis=1, keepdims=True)
    csq = jnp.sum(code_book ** 2, axis=1).reshape(1, K)
    idx_col = _tc_argmin_call(x.astype(jnp.bfloat16), xsq, csq, code_book)
    sums = _sc_quantize_sum(code_book, idx_col.reshape(N))
    return _tc_finalize_call(sums, history_item_masks)
